# initial kernel scaffold (unmeasured)
import jax
import jax.numpy as jnp
from jax import lax
from jax.experimental import pallas as pl
from jax.experimental.pallas import tpu as pltpu

N_DEV = 8


def kernel(A, B):
    M, K = A.shape
    N = B.shape[1]
    m_per = M // N_DEV

    a16 = A.astype(jnp.bfloat16)
    b16 = B.astype(jnp.bfloat16)

    def body(a_ref, b_ref, out_ref, acc_ref, send_sems, recv_sems, credit_sem):
        my = lax.axis_index("i")
        left = lax.rem(my - 1 + N_DEV, N_DEV)
        right = lax.rem(my + 1, N_DEV)

        barrier_sem = pltpu.get_barrier_semaphore()
        for nbr in (left, right):
            pl.semaphore_signal(
                barrier_sem, inc=1,
                device_id=(nbr,), device_id_type=pl.DeviceIdType.MESH,
            )
        pl.semaphore_wait(barrier_sem, 2)

        def partial_for(c):
            return jnp.dot(
                a_ref[pl.ds(c * m_per, m_per), :], b_ref[...],
                preferred_element_type=jnp.float32,
            )

        c0 = lax.rem(my - 1 + N_DEV, N_DEV)
        acc_ref[0] = partial_for(c0)

        for s in range(N_DEV - 1):
            src_slot = s % 2
            dst_slot = (s + 1) % 2
            if s >= 2:
                pl.semaphore_wait(credit_sem, 1)
            rdma = pltpu.make_async_remote_copy(
                src_ref=acc_ref.at[src_slot],
                dst_ref=acc_ref.at[dst_slot],
                send_sem=send_sems.at[src_slot],
                recv_sem=recv_sems.at[dst_slot],
                device_id=(right,),
                device_id_type=pl.DeviceIdType.MESH,
            )
            rdma.start()
            rdma.wait()
            if 1 <= s <= N_DEV - 3:
                pl.semaphore_signal(
                    credit_sem, inc=1,
                    device_id=(left,), device_id_type=pl.DeviceIdType.MESH,
                )
            c = lax.rem(my - s - 2 + 2 * N_DEV, N_DEV)
            if s == N_DEV - 2:
                out_ref[...] = acc_ref[dst_slot] + partial_for(c)
            else:
                acc_ref[dst_slot] = acc_ref[dst_slot] + partial_for(c)

    return pl.pallas_call(
        body,
        out_shape=jax.ShapeDtypeStruct((m_per, N), jnp.float32),
        in_specs=[
            pl.BlockSpec(memory_space=pltpu.VMEM),
            pl.BlockSpec(memory_space=pltpu.VMEM),
        ],
        out_specs=pl.BlockSpec(memory_space=pltpu.VMEM),
        scratch_shapes=[
            pltpu.VMEM((2, m_per, N), jnp.float32),
            pltpu.SemaphoreType.DMA((2,)),
            pltpu.SemaphoreType.DMA((2,)),
            pltpu.SemaphoreType.REGULAR,
        ],
        compiler_params=pltpu.CompilerParams(collective_id=0),
    )(a16, b16)


# baseline (device time: 782107 ns/iter reference)
import jax

jax.config.update("jax_compilation_cache_dir", "/tmp/jaxcache")
jax.config.update("jax_persistent_cache_min_compile_time_secs", 1.0)
jax.config.update("jax_persistent_cache_min_entry_size_bytes", 0)

import jax.numpy as jnp
from jax import lax
from jax.experimental import pallas as pl
from jax.experimental.pallas import tpu as pltpu

N_DEV = 8


def kernel(A, B):
    M, K = A.shape
    N = B.shape[1]
    m_per = M // N_DEV

    a16 = A.astype(jnp.bfloat16)
    b16 = B.astype(jnp.bfloat16)

    def body(a_hbm, b_ref, out_ref, acc_ref, a_buf, send_sems, recv_sems,
             copy_sem, credit_sem):
        my = lax.axis_index("i")
        left = lax.rem(my - 1 + N_DEV, N_DEV)
        right = lax.rem(my + 1, N_DEV)

        barrier_sem = pltpu.get_barrier_semaphore()
        for nbr in (left, right):
            pl.semaphore_signal(
                barrier_sem, inc=1,
                device_id=(nbr,), device_id_type=pl.DeviceIdType.MESH,
            )
        pl.semaphore_wait(barrier_sem, 2)

        def fetch_a(c):
            cp = pltpu.make_async_copy(
                a_hbm.at[pl.ds(c * m_per, m_per), :], a_buf, copy_sem,
            )
            cp.start()
            cp.wait()

        def partial():
            return jnp.dot(a_buf[...], b_ref[...],
                           preferred_element_type=jnp.float32)

        fetch_a(lax.rem(my - 1 + N_DEV, N_DEV))
        acc_ref[0] = partial()

        for s in range(N_DEV - 1):
            src_slot = s % 2
            dst_slot = (s + 1) % 2
            if s >= 2:
                pl.semaphore_wait(credit_sem, 1)
            rdma = pltpu.make_async_remote_copy(
                src_ref=acc_ref.at[src_slot],
                dst_ref=acc_ref.at[dst_slot],
                send_sem=send_sems.at[src_slot],
                recv_sem=recv_sems.at[dst_slot],
                device_id=(right,),
                device_id_type=pl.DeviceIdType.MESH,
            )
            rdma.start()
            rdma.wait()
            if 1 <= s <= N_DEV - 3:
                pl.semaphore_signal(
                    credit_sem, inc=1,
                    device_id=(left,), device_id_type=pl.DeviceIdType.MESH,
                )
            fetch_a(lax.rem(my - s - 2 + 2 * N_DEV, N_DEV))
            if s == N_DEV - 2:
                out_ref[...] = acc_ref[dst_slot] + partial()
            else:
                acc_ref[dst_slot] = acc_ref[dst_slot] + partial()

    return pl.pallas_call(
        body,
        out_shape=jax.ShapeDtypeStruct((m_per, N), jnp.float32),
        in_specs=[
            pl.BlockSpec(memory_space=pl.ANY),
            pl.BlockSpec(memory_space=pltpu.VMEM),
        ],
        out_specs=pl.BlockSpec(memory_space=pltpu.VMEM),
        scratch_shapes=[
            pltpu.VMEM((2, m_per, N), jnp.float32),
            pltpu.VMEM((m_per, K), jnp.bfloat16),
            pltpu.SemaphoreType.DMA((2,)),
            pltpu.SemaphoreType.DMA((2,)),
            pltpu.SemaphoreType.DMA,
            pltpu.SemaphoreType.REGULAR,
        ],
        compiler_params=pltpu.CompilerParams(
            collective_id=0,
            vmem_limit_bytes=48 * 1024 * 1024,
        ),
    )(a16, b16)
